# baseline (speedup 1.0000x reference)
import functools

import jax
import jax.numpy as jnp
from jax import lax
from jax.experimental import pallas as pl
from jax.experimental.pallas import tpu as pltpu
from jax.experimental.pallas import tpu_sc as plsc

N_NODES = 10000
N_EDGES = 320000
F_IN = 128
NH = 8
DH = 16
FO = NH * DH

NC = 2
NS = 16
NW = NC * NS

FW = FO + DH
CHUNK = 224
CPW = 45
NCHUNKS = NW * CPW
E_PAD = NCHUNKS * CHUNK
NROWS = N_NODES + 8
RPT = 624
RPT_LAST = N_NODES - 15 * RPT
ZR_LAST = NROWS - 15 * RPT

_BR = 1000


def _tc_prologue(x, W, Acat):
    def body(x_ref, w_ref, a_ref, h_ref, t_ref):
        h = jnp.dot(x_ref[...], w_ref[...],
                    preferred_element_type=jnp.float32,
                    precision=lax.Precision.HIGHEST)
        h_ref[...] = jnp.pad(h, ((0, 0), (0, DH)))
        t_ref[...] = jnp.dot(h, a_ref[...],
                             preferred_element_type=jnp.float32,
                             precision=lax.Precision.HIGHEST)

    return pl.pallas_call(
        body,
        grid=(N_NODES // _BR,),
        in_specs=[
            pl.BlockSpec((_BR, F_IN), lambda i: (i, 0)),
            pl.BlockSpec((F_IN, FO), lambda i: (0, 0)),
            pl.BlockSpec((F_IN, 32), lambda i: (0, 0)),
        ],
        out_specs=[
            pl.BlockSpec((_BR, FW), lambda i: (i, 0)),
            pl.BlockSpec((_BR, 32), lambda i: (i, 0)),
        ],
        out_shape=[
            jax.ShapeDtypeStruct((N_NODES, FW), jnp.float32),
            jax.ShapeDtypeStruct((N_NODES, 32), jnp.float32),
        ],
    )(x, W, Acat)


def _sc_edge_pass(h, tsrc, tdst, src3, dst3, z):
    mesh = plsc.VectorSubcoreMesh(core_axis_name="c", subcore_axis_name="s")

    @functools.partial(
        pl.kernel,
        out_type=jax.ShapeDtypeStruct((NC, N_NODES, FW), jnp.float32),
        mesh=mesh,
        scratch_types=[
            pltpu.VMEM_SHARED((NROWS, FW), jnp.float32),
            pltpu.VMEM((1, CHUNK), jnp.int32),
            pltpu.VMEM((1, CHUNK), jnp.int32),
            pltpu.VMEM((CHUNK, DH), jnp.float32),
            pltpu.VMEM((CHUNK, DH), jnp.float32),
            pltpu.VMEM((CHUNK, FW), jnp.float32),
            pltpu.SemaphoreType.DMA,
        ],
        compiler_params=pltpu.CompilerParams(use_tc_tiling_on_sc=False),
    )
    def k(h_hbm, ts_hbm, td_hbm, src_hbm, dst_hbm, z_hbm,
          pout_hbm,
          acc, src_v, dst_v, a_s, a_d, hrows, sem):
        cid = lax.axis_index("c")
        sid = lax.axis_index("s")
        wid = sid * NC + cid
        lo = sid * RPT

        @pl.when(sid < NS - 1)
        def _():
            rows = pl.ds(lo, RPT)
            pltpu.sync_copy(z_hbm.at[rows], acc.at[rows])

        @pl.when(sid == NS - 1)
        def _():
            rows = pl.ds(15 * RPT, ZR_LAST)
            pltpu.sync_copy(z_hbm.at[rows], acc.at[rows])

        plsc.subcore_barrier()

        def chunk_body(j, carry):
            gcid = wid + NW * j
            pltpu.sync_copy(src_hbm.at[gcid], src_v)
            pltpu.sync_copy(dst_hbm.at[gcid], dst_v)
            c1 = pltpu.async_copy(ts_hbm.at[src_v.at[0]], a_s, sem)
            c2 = pltpu.async_copy(td_hbm.at[dst_v.at[0]], a_d, sem)
            c3 = pltpu.async_copy(h_hbm.at[src_v.at[0]], hrows, sem)
            c1.wait()
            c2.wait()

            def prow(r, c):
                v = a_s[r] + a_d[r]
                a_s[r] = jnp.exp(jnp.where(v >= 0.0, v, 0.2 * v))
                return c

            lax.fori_loop(0, CHUNK, prow, None, unroll=4)
            c3.wait()

            def srow(e, c):
                pr = a_s[e]
                for hh in range(NH):
                    seg = pl.ds(hh * DH, DH)
                    hrows[e, seg] = hrows[e, seg] * pr[hh]
                hrows[e, pl.ds(FO, DH)] = pr
                return c

            lax.fori_loop(0, CHUNK, srow, None, unroll=2)

            pltpu.async_copy(hrows, acc.at[dst_v.at[0]], sem, add=True).wait()
            return carry

        lax.fori_loop(0, CPW, chunk_body, None)
        plsc.subcore_barrier()

        @pl.when(sid < NS - 1)
        def _():
            rows = pl.ds(lo, RPT)
            pltpu.sync_copy(acc.at[rows], pout_hbm.at[cid, rows])

        @pl.when(sid == NS - 1)
        def _():
            rows = pl.ds(15 * RPT, RPT_LAST)
            pltpu.sync_copy(acc.at[rows], pout_hbm.at[cid, rows])

    return k(h, tsrc, tdst, src3, dst3, z)


def _tc_epilogue(p0, p1, K16):
    def body(p0_ref, p1_ref, k_ref, o_ref):
        acc = p0_ref[...] + p1_ref[...]
        r = 1.0 / (acc[:, FO:FW] + 1e-10)
        o_ref[...] = acc[:, 0:FO] * jnp.dot(r, k_ref[...],
                                            preferred_element_type=jnp.float32)

    return pl.pallas_call(
        body,
        grid=(N_NODES // _BR,),
        in_specs=[
            pl.BlockSpec((_BR, FW), lambda i: (i, 0)),
            pl.BlockSpec((_BR, FW), lambda i: (i, 0)),
            pl.BlockSpec((DH, FO), lambda i: (0, 0)),
        ],
        out_specs=pl.BlockSpec((_BR, FO), lambda i: (i, 0)),
        out_shape=jax.ShapeDtypeStruct((N_NODES, FO), jnp.float32),
    )(p0, p1, K16)


@jax.jit
def kernel(x, edge_index, W, a_src, a_dst):
    f = jnp.float32
    rows = jnp.arange(F_IN)
    cols = rows // DH
    As = jnp.zeros((F_IN, NH), f).at[rows, cols].set(a_src.reshape(-1))
    Ad = jnp.zeros((F_IN, NH), f).at[rows, cols].set(a_dst.reshape(-1))
    zpad = jnp.zeros((F_IN, NH), f)
    Acat = jnp.concatenate([As, zpad, Ad, zpad], axis=1)

    h, T = _tc_prologue(x, W, Acat)
    tsrc = jnp.pad(T[:, 0:16], ((0, NROWS - N_NODES), (0, 0)))
    tdst = jnp.pad(T[:, 16:32], ((0, NROWS - N_NODES), (0, 0)))

    npad = E_PAD - N_EDGES
    src3 = jnp.concatenate(
        [edge_index[0], jnp.zeros((npad,), jnp.int32)]).reshape(NCHUNKS, 1, CHUNK)
    dst3 = jnp.concatenate(
        [edge_index[1], jnp.full((npad,), N_NODES, jnp.int32)]).reshape(NCHUNKS, 1, CHUNK)
    z = jnp.zeros((NROWS, FW), f)

    pout = _sc_edge_pass(h, tsrc, tdst, src3, dst3, z)

    K8 = jnp.repeat(jnp.eye(NH, dtype=f), DH, axis=1)
    K16 = jnp.concatenate([K8, jnp.zeros((NH, FO), f)], axis=0)
    return _tc_epilogue(pout[0], pout[1], K16)


# speedup vs baseline: 1.2350x; 1.2350x over previous
import functools

import jax
import jax.numpy as jnp
from jax import lax
from jax.experimental import pallas as pl
from jax.experimental.pallas import tpu as pltpu
from jax.experimental.pallas import tpu_sc as plsc

N_NODES = 10000
N_EDGES = 320000
F_IN = 128
NH = 8
DH = 16
FO = NH * DH

NC = 2
NS = 16
NW = NC * NS

CHUNK = 240
CPW = 42
NCHUNKS = NW * CPW
E_PAD = NCHUNKS * CHUNK
NROWS = N_NODES + 8
RPT = 624
RPT_LAST = N_NODES - 15 * RPT
ZR_LAST = NROWS - 15 * RPT

_BR = 1000


def _tc_prologue(x, W, Acat):
    def body(x_ref, w_ref, a_ref, h_ref, t_ref):
        h = jnp.dot(x_ref[...], w_ref[...],
                    preferred_element_type=jnp.float32,
                    precision=lax.Precision.HIGHEST)
        h_ref[...] = h
        t_ref[...] = jnp.dot(h, a_ref[...],
                             preferred_element_type=jnp.float32,
                             precision=lax.Precision.HIGHEST)

    return pl.pallas_call(
        body,
        grid=(N_NODES // _BR,),
        in_specs=[
            pl.BlockSpec((_BR, F_IN), lambda i: (i, 0)),
            pl.BlockSpec((F_IN, FO), lambda i: (0, 0)),
            pl.BlockSpec((F_IN, 32), lambda i: (0, 0)),
        ],
        out_specs=[
            pl.BlockSpec((_BR, FO), lambda i: (i, 0)),
            pl.BlockSpec((_BR, 32), lambda i: (i, 0)),
        ],
        out_shape=[
            jax.ShapeDtypeStruct((N_NODES, FO), jnp.float32),
            jax.ShapeDtypeStruct((N_NODES, 32), jnp.float32),
        ],
    )(x, W, Acat)


def _sc_edge_pass(h, t2, e2, z128, z16):
    mesh = plsc.VectorSubcoreMesh(core_axis_name="c", subcore_axis_name="s")

    @functools.partial(
        pl.kernel,
        out_type=[
            jax.ShapeDtypeStruct((NC, N_NODES, FO), jnp.float32),
            jax.ShapeDtypeStruct((NC, N_NODES, DH), jnp.float32),
        ],
        mesh=mesh,
        scratch_types=[
            pltpu.VMEM_SHARED((NROWS, FO), jnp.float32),
            pltpu.VMEM_SHARED((NROWS, DH), jnp.float32),
            pltpu.VMEM((2, CHUNK), jnp.int32),
            pltpu.VMEM((2, CHUNK), jnp.int32),
            pltpu.VMEM((1, 2 * CHUNK), jnp.int32),
            pltpu.VMEM((2 * CHUNK, DH), jnp.float32),
            pltpu.VMEM((CHUNK, FO), jnp.float32),
            pltpu.SemaphoreType.DMA,
            pltpu.SemaphoreType.DMA,
            pltpu.SemaphoreType.DMA,
        ],
        compiler_params=pltpu.CompilerParams(use_tc_tiling_on_sc=False),
    )
    def k(h_hbm, t2_hbm, e2_hbm, z128_hbm, z16_hbm,
          pout_hbm, sout_hbm,
          acc, sacc, idxa, idxb, idxg, a_sd, hrows, semG, semI, semS):
        cid = lax.axis_index("c")
        sid = lax.axis_index("s")
        wid = sid * NC + cid
        lo = sid * RPT
        idx = (idxa, idxb)

        @pl.when(sid < NS - 1)
        def _():
            rows = pl.ds(lo, RPT)
            pltpu.sync_copy(z128_hbm.at[rows], acc.at[rows])
            pltpu.sync_copy(z16_hbm.at[rows], sacc.at[rows])

        @pl.when(sid == NS - 1)
        def _():
            rows = pl.ds(15 * RPT, ZR_LAST)
            pltpu.sync_copy(z128_hbm.at[rows], acc.at[rows])
            pltpu.sync_copy(z16_hbm.at[rows], sacc.at[rows])

        plsc.subcore_barrier()

        pltpu.async_copy(e2_hbm.at[wid], idxa, semI)

        def pair_body(jj, carry):
            for b in (0, 1):
                j = 2 * jj + b
                nb = 1 - b
                pltpu.make_async_copy(e2_hbm.at[0], idx[b], semI).wait()

                for l in range(CHUNK // 16):
                    s16 = pl.ds(16 * l, 16)
                    idxg[0, s16] = idx[b][0, s16]
                    idxg[0, pl.ds(CHUNK + 16 * l, 16)] = idx[b][1, s16] + NROWS

                c1 = pltpu.async_copy(t2_hbm.at[idxg.at[0]], a_sd, semG)
                c2 = pltpu.async_copy(h_hbm.at[idx[b].at[0]], hrows, semG)
                c1.wait()

                def prow(r, c):
                    v = a_sd[r] + a_sd[CHUNK + r]
                    a_sd[r] = jnp.exp(jnp.where(v >= 0.0, v, 0.2 * v))
                    return c

                lax.fori_loop(0, CHUNK, prow, None, unroll=4)
                c2.wait()

                def srow(e, c):
                    pr = a_sd[e]
                    for hh in range(NH):
                        seg = pl.ds(hh * DH, DH)
                        hrows[e, seg] = hrows[e, seg] * pr[hh]
                    return c

                lax.fori_loop(0, CHUNK, srow, None, unroll=2)

                s1 = pltpu.async_copy(a_sd.at[pl.ds(0, CHUNK)],
                                      sacc.at[idx[b].at[1]], semS, add=True)
                s2 = pltpu.async_copy(hrows, acc.at[idx[b].at[1]], semS,
                                      add=True)

                @pl.when(j + 1 < CPW)
                def _():
                    pltpu.async_copy(e2_hbm.at[wid + NW * (j + 1)], idx[nb],
                                     semI)

                s1.wait()
                s2.wait()
            return carry

        lax.fori_loop(0, CPW // 2, pair_body, None)
        plsc.subcore_barrier()

        @pl.when(sid < NS - 1)
        def _():
            rows = pl.ds(lo, RPT)
            pltpu.sync_copy(acc.at[rows], pout_hbm.at[cid, rows])
            pltpu.sync_copy(sacc.at[rows], sout_hbm.at[cid, rows])

        @pl.when(sid == NS - 1)
        def _():
            rows = pl.ds(15 * RPT, RPT_LAST)
            pltpu.sync_copy(acc.at[rows], pout_hbm.at[cid, rows])
            pltpu.sync_copy(sacc.at[rows], sout_hbm.at[cid, rows])

    return k(h, t2, e2, z128, z16)


def _tc_epilogue(p0, p1, s0, s1, K16):
    def body(p0_ref, p1_ref, s0_ref, s1_ref, k_ref, o_ref):
        acc = p0_ref[...] + p1_ref[...]
        r = 1.0 / (s0_ref[...] + s1_ref[...] + 1e-10)
        o_ref[...] = acc * jnp.dot(r, k_ref[...],
                                   preferred_element_type=jnp.float32)

    return pl.pallas_call(
        body,
        grid=(N_NODES // _BR,),
        in_specs=[
            pl.BlockSpec((_BR, FO), lambda i: (i, 0)),
            pl.BlockSpec((_BR, FO), lambda i: (i, 0)),
            pl.BlockSpec((_BR, DH), lambda i: (i, 0)),
            pl.BlockSpec((_BR, DH), lambda i: (i, 0)),
            pl.BlockSpec((DH, FO), lambda i: (0, 0)),
        ],
        out_specs=pl.BlockSpec((_BR, FO), lambda i: (i, 0)),
        out_shape=jax.ShapeDtypeStruct((N_NODES, FO), jnp.float32),
    )(p0, p1, s0, s1, K16)


@jax.jit
def kernel(x, edge_index, W, a_src, a_dst):
    f = jnp.float32
    rows = jnp.arange(F_IN)
    cols = rows // DH
    As = jnp.zeros((F_IN, NH), f).at[rows, cols].set(a_src.reshape(-1))
    Ad = jnp.zeros((F_IN, NH), f).at[rows, cols].set(a_dst.reshape(-1))
    zpad = jnp.zeros((F_IN, NH), f)
    Acat = jnp.concatenate([As, zpad, Ad, zpad], axis=1)

    h, T = _tc_prologue(x, W, Acat)
    tsrc = jnp.pad(T[:, 0:16], ((0, NROWS - N_NODES), (0, 0)))
    tdst = jnp.pad(T[:, 16:32], ((0, NROWS - N_NODES), (0, 0)))
    t2 = jnp.concatenate([tsrc, tdst], axis=0)

    npad = E_PAD - N_EDGES
    srcp = jnp.concatenate([edge_index[0], jnp.zeros((npad,), jnp.int32)])
    dstp = jnp.concatenate([edge_index[1], jnp.full((npad,), N_NODES, jnp.int32)])
    e2 = jnp.stack([srcp.reshape(NCHUNKS, CHUNK),
                    dstp.reshape(NCHUNKS, CHUNK)], axis=1)
    z128 = jnp.zeros((NROWS, FO), f)
    z16 = jnp.zeros((NROWS, DH), f)

    pout, sout = _sc_edge_pass(h, t2, e2, z128, z16)

    K8 = jnp.repeat(jnp.eye(NH, dtype=f), DH, axis=1)
    K16 = jnp.concatenate([K8, jnp.zeros((NH, FO), f)], axis=0)
    return _tc_epilogue(pout[0], pout[1], sout[0], sout[1], K16)


# speedup vs baseline: 1.2380x; 1.0024x over previous
import functools

import jax
import jax.numpy as jnp
from jax import lax
from jax.experimental import pallas as pl
from jax.experimental.pallas import tpu as pltpu
from jax.experimental.pallas import tpu_sc as plsc

N_NODES = 10000
N_EDGES = 320000
F_IN = 128
NH = 8
DH = 16
FO = NH * DH

NC = 2
NS = 16
NW = NC * NS

CHUNK = 240
CPW = 42
NCHUNKS = NW * CPW
E_PAD = NCHUNKS * CHUNK
NROWS = N_NODES + 8
RPT = 624
RPT_LAST = N_NODES - 15 * RPT
ZR_LAST = NROWS - 15 * RPT

_BR = 1000


def _tc_prologue(x, W, Acat):
    def body(x_ref, w_ref, a_ref, h_ref, t_ref):
        h = jnp.dot(x_ref[...], w_ref[...],
                    preferred_element_type=jnp.float32,
                    precision=lax.Precision.HIGHEST)
        h_ref[...] = h
        t_ref[...] = jnp.dot(h, a_ref[...],
                             preferred_element_type=jnp.float32,
                             precision=lax.Precision.HIGHEST)

    return pl.pallas_call(
        body,
        grid=(N_NODES // _BR,),
        in_specs=[
            pl.BlockSpec((_BR, F_IN), lambda i: (i, 0)),
            pl.BlockSpec((F_IN, FO), lambda i: (0, 0)),
            pl.BlockSpec((F_IN, 32), lambda i: (0, 0)),
        ],
        out_specs=[
            pl.BlockSpec((_BR, FO), lambda i: (i, 0)),
            pl.BlockSpec((_BR, 32), lambda i: (i, 0)),
        ],
        out_shape=[
            jax.ShapeDtypeStruct((N_NODES, FO), jnp.float32),
            jax.ShapeDtypeStruct((N_NODES, 32), jnp.float32),
        ],
    )(x, W, Acat)


def _sc_edge_pass(h, t2, e2, z128, z16):
    mesh = plsc.VectorSubcoreMesh(core_axis_name="c", subcore_axis_name="s")

    @functools.partial(
        pl.kernel,
        out_type=[
            jax.ShapeDtypeStruct((NC, N_NODES, FO), jnp.float32),
            jax.ShapeDtypeStruct((NC, N_NODES, DH), jnp.float32),
        ],
        mesh=mesh,
        scratch_types=[
            pltpu.VMEM_SHARED((NROWS, FO), jnp.float32),
            pltpu.VMEM_SHARED((NROWS, DH), jnp.float32),
            pltpu.VMEM((2, CHUNK), jnp.int32),
            pltpu.VMEM((2, CHUNK), jnp.int32),
            pltpu.VMEM((1, 2 * CHUNK), jnp.int32),
            pltpu.VMEM((2 * CHUNK, DH), jnp.float32),
            pltpu.VMEM((CHUNK, FO), jnp.float32),
            pltpu.SemaphoreType.DMA,
            pltpu.SemaphoreType.DMA,
            pltpu.SemaphoreType.DMA,
        ],
        compiler_params=pltpu.CompilerParams(use_tc_tiling_on_sc=False),
    )
    def k(h_hbm, t2_hbm, e2_hbm, z128_hbm, z16_hbm,
          pout_hbm, sout_hbm,
          acc, sacc, idxa, idxb, idxg, a_sd, hrows, semG, semI, semS):
        cid = lax.axis_index("c")
        sid = lax.axis_index("s")
        wid = sid * NC + cid
        lo = sid * RPT
        idx = (idxa, idxb)

        @pl.when(sid < NS - 1)
        def _():
            rows = pl.ds(lo, RPT)
            pltpu.sync_copy(z128_hbm.at[rows], acc.at[rows])
            pltpu.sync_copy(z16_hbm.at[rows], sacc.at[rows])

        @pl.when(sid == NS - 1)
        def _():
            rows = pl.ds(15 * RPT, ZR_LAST)
            pltpu.sync_copy(z128_hbm.at[rows], acc.at[rows])
            pltpu.sync_copy(z16_hbm.at[rows], sacc.at[rows])

        plsc.subcore_barrier()

        pltpu.async_copy(e2_hbm.at[wid], idxa, semI)

        def pair_body(jj, carry):
            for b in (0, 1):
                j = 2 * jj + b
                nb = 1 - b
                pltpu.make_async_copy(e2_hbm.at[0], idx[b], semI).wait()

                @pl.when(j + 1 < CPW)
                def _():
                    pltpu.async_copy(e2_hbm.at[wid + NW * (j + 1)], idx[nb],
                                     semI)

                for l in range(CHUNK // 16):
                    s16 = pl.ds(16 * l, 16)
                    idxg[0, s16] = idx[b][0, s16]
                    idxg[0, pl.ds(CHUNK + 16 * l, 16)] = idx[b][1, s16] + NROWS

                c1 = pltpu.async_copy(t2_hbm.at[idxg.at[0]], a_sd, semG)
                c2 = pltpu.async_copy(h_hbm.at[idx[b].at[0]], hrows, semG)
                c1.wait()

                def prow(r, c):
                    v = a_sd[r] + a_sd[CHUNK + r]
                    a_sd[r] = jnp.exp(jnp.where(v >= 0.0, v, 0.2 * v))
                    return c

                lax.fori_loop(0, CHUNK, prow, None, unroll=8)

                s1 = pltpu.async_copy(a_sd.at[pl.ds(0, CHUNK)],
                                      sacc.at[idx[b].at[1]], semS, add=True)
                c2.wait()

                def srow(e, c):
                    pr = a_sd[e]
                    for hh in range(NH):
                        seg = pl.ds(hh * DH, DH)
                        hrows[e, seg] = hrows[e, seg] * pr[hh]
                    return c

                lax.fori_loop(0, CHUNK, srow, None, unroll=4)

                s2 = pltpu.async_copy(hrows, acc.at[idx[b].at[1]], semS,
                                      add=True)
                s1.wait()
                s2.wait()
            return carry

        lax.fori_loop(0, CPW // 2, pair_body, None)
        plsc.subcore_barrier()

        @pl.when(sid < NS - 1)
        def _():
            rows = pl.ds(lo, RPT)
            pltpu.sync_copy(acc.at[rows], pout_hbm.at[cid, rows])
            pltpu.sync_copy(sacc.at[rows], sout_hbm.at[cid, rows])

        @pl.when(sid == NS - 1)
        def _():
            rows = pl.ds(15 * RPT, RPT_LAST)
            pltpu.sync_copy(acc.at[rows], pout_hbm.at[cid, rows])
            pltpu.sync_copy(sacc.at[rows], sout_hbm.at[cid, rows])

    return k(h, t2, e2, z128, z16)


def _tc_epilogue(p0, p1, s0, s1, K16):
    def body(p0_ref, p1_ref, s0_ref, s1_ref, k_ref, o_ref):
        acc = p0_ref[...] + p1_ref[...]
        r = 1.0 / (s0_ref[...] + s1_ref[...] + 1e-10)
        o_ref[...] = acc * jnp.dot(r, k_ref[...],
                                   preferred_element_type=jnp.float32)

    return pl.pallas_call(
        body,
        grid=(N_NODES // _BR,),
        in_specs=[
            pl.BlockSpec((_BR, FO), lambda i: (i, 0)),
            pl.BlockSpec((_BR, FO), lambda i: (i, 0)),
            pl.BlockSpec((_BR, DH), lambda i: (i, 0)),
            pl.BlockSpec((_BR, DH), lambda i: (i, 0)),
            pl.BlockSpec((DH, FO), lambda i: (0, 0)),
        ],
        out_specs=pl.BlockSpec((_BR, FO), lambda i: (i, 0)),
        out_shape=jax.ShapeDtypeStruct((N_NODES, FO), jnp.float32),
    )(p0, p1, s0, s1, K16)


@jax.jit
def kernel(x, edge_index, W, a_src, a_dst):
    f = jnp.float32
    rows = jnp.arange(F_IN)
    cols = rows // DH
    As = jnp.zeros((F_IN, NH), f).at[rows, cols].set(a_src.reshape(-1))
    Ad = jnp.zeros((F_IN, NH), f).at[rows, cols].set(a_dst.reshape(-1))
    zpad = jnp.zeros((F_IN, NH), f)
    Acat = jnp.concatenate([As, zpad, Ad, zpad], axis=1)

    h, T = _tc_prologue(x, W, Acat)
    tsrc = jnp.pad(T[:, 0:16], ((0, NROWS - N_NODES), (0, 0)))
    tdst = jnp.pad(T[:, 16:32], ((0, NROWS - N_NODES), (0, 0)))
    t2 = jnp.concatenate([tsrc, tdst], axis=0)

    npad = E_PAD - N_EDGES
    srcp = jnp.concatenate([edge_index[0], jnp.zeros((npad,), jnp.int32)])
    dstp = jnp.concatenate([edge_index[1], jnp.full((npad,), N_NODES, jnp.int32)])
    e2 = jnp.stack([srcp.reshape(NCHUNKS, CHUNK),
                    dstp.reshape(NCHUNKS, CHUNK)], axis=1)
    z128 = jnp.zeros((NROWS, FO), f)
    z16 = jnp.zeros((NROWS, DH), f)

    pout, sout = _sc_edge_pass(h, t2, e2, z128, z16)

    K8 = jnp.repeat(jnp.eye(NH, dtype=f), DH, axis=1)
    K16 = jnp.concatenate([K8, jnp.zeros((NH, FO), f)], axis=0)
    return _tc_epilogue(pout[0], pout[1], sout[0], sout[1], K16)
